# baseline (speedup 1.0000x reference)
import functools

import jax
import jax.numpy as jnp
from jax import lax
from jax.experimental import pallas as pl
from jax.experimental.pallas import tpu as pltpu
from jax.experimental.pallas import tpu_sc as plsc

N_NODES = 10000
N_EDGES = 320000
D = 128
D_OUT = 2

NC = 2
NS = 16
NW = NC * NS
EPW = N_EDGES // NW
CHUNK = 80
NCHUNK = EPW // CHUNK
GROUPS = CHUNK // 16

_LANE = None


def _vshuffle(v, idx):
    return lax.gather(
        v, idx[:, None],
        dimension_numbers=lax.GatherDimensionNumbers(
            offset_dims=(), collapsed_slice_dims=(0,), start_index_map=(0,)),
        slice_sizes=(1,),
        mode=lax.GatherScatterMode.PROMISE_IN_BOUNDS)


def _pack_bf16(r):
    rb = r.astype(jnp.bfloat16)
    lo = lax.bitcast_convert_type(rb[:, :D // 2], jnp.uint16).astype(jnp.uint32)
    hi = lax.bitcast_convert_type(rb[:, D // 2:], jnp.uint16).astype(jnp.uint32)
    return lax.bitcast_convert_type(lo | (hi << 16), jnp.int32)


def _node_tables_body(x_ref, ws_ref, wd_ref, xs_ref, xd_ref):
    xb = x_ref[...]
    xs_ref[...] = jnp.dot(xb, ws_ref[...], preferred_element_type=jnp.float32)
    xd_ref[...] = jnp.dot(xb, wd_ref[...], preferred_element_type=jnp.float32)


def _node_tables(x, ws, wd):
    bn = 2000
    grid = N_NODES // bn
    return pl.pallas_call(
        _node_tables_body,
        grid=(grid,),
        in_specs=[
            pl.BlockSpec((bn, D), lambda i: (i, 0)),
            pl.BlockSpec((D, D), lambda i: (0, 0)),
            pl.BlockSpec((D, D), lambda i: (0, 0)),
        ],
        out_specs=[
            pl.BlockSpec((bn, D), lambda i: (i, 0)),
            pl.BlockSpec((bn, D), lambda i: (i, 0)),
        ],
        out_shape=[
            jax.ShapeDtypeStruct((N_NODES, D), jnp.float32),
            jax.ShapeDtypeStruct((N_NODES, D), jnp.float32),
        ],
    )(x, ws, wd)


def _edge_hidden_body(ea_ref, wa_ref, b_ref, a_ref):
    ea2 = ea_ref[...]
    h0 = jnp.dot(ea2[:, :D], wa_ref[...],
                 preferred_element_type=jnp.float32) + b_ref[...]
    h1 = jnp.dot(ea2[:, D:], wa_ref[...],
                 preferred_element_type=jnp.float32) + b_ref[...]
    a_ref[...] = jnp.concatenate([_pack_bf16(h0), _pack_bf16(h1)], axis=1)


def _edge_hidden(edge_attr2, wa, be1_row):
    bm = 2000
    nrows = edge_attr2.shape[0]
    grid = nrows // bm
    return pl.pallas_call(
        _edge_hidden_body,
        grid=(grid,),
        in_specs=[
            pl.BlockSpec((bm, 2 * D), lambda i: (i, 0)),
            pl.BlockSpec((D, D), lambda i: (0, 0)),
            pl.BlockSpec((1, D), lambda i: (0, 0)),
        ],
        out_specs=pl.BlockSpec((bm, D), lambda i: (i, 0)),
        out_shape=jax.ShapeDtypeStruct((nrows, D), jnp.int32),
    )(edge_attr2, wa, be1_row)


def _make_sc_edge_body(epw, nchunk):
  def _sc_edge_body(xs_hbm, xd_hbm, a_hbm, src_hbm, dst_hbm, w2_hbm, be2_hbm,
                  out_hbm,
                  si_all, di_all, gs0, gs1, gd0, gd1,
                  av0, av1,
                  t_v, w2_v, be2_v, sem_d0, sem_d1):
    wid = lax.axis_index("s") * NC + lax.axis_index("c")
    base = wid * epw
    gs = (gs0, gs1)
    gd = (gd0, gd1)
    av = (av0, av1)
    semd = (sem_d0, sem_d1)

    pltpu.sync_copy(src_hbm.at[pl.ds(base, epw)], si_all)
    pltpu.sync_copy(dst_hbm.at[pl.ds(base, epw)], di_all)
    pltpu.sync_copy(w2_hbm, w2_v)
    pltpu.sync_copy(be2_hbm, be2_v)
    be2 = be2_v[...][0]
    w2regs = [w2_v[pl.ds(r * 16, 16)] for r in range(8)]

    lane = lax.iota(jnp.int32, 16)
    zeros16 = jnp.zeros((16,), jnp.float32)

    def zero_body(i, carry):
        t_v[pl.ds(i * 16, 16)] = zeros16
        return carry
    lax.fori_loop(0, N_NODES // 16, zero_body, 0)

    def dat_copies(c, b):
        loc = c * CHUNK
        aoff = wid * (epw // 2) + c * (CHUNK // 2)
        return (
            pltpu.make_async_copy(xs_hbm.at[si_all.at[pl.ds(loc, CHUNK)]],
                                  gs[b], semd[b]),
            pltpu.make_async_copy(xd_hbm.at[di_all.at[pl.ds(loc, CHUNK)]],
                                  gd[b], semd[b]),
            pltpu.make_async_copy(a_hbm.at[pl.ds(aoff, CHUNK // 2)],
                                  av[b], semd[b]),
        )

    def compute(c, b):
        gsb, gdb, avb = gs[b], gd[b], av[b]

        def half_body(i, h_prev):
            accs = []
            for e in range(8):
                row = i * 8 + e
                arow = i * 4 + e // 2
                abase = (e % 2) * 64
                acc = None
                for r in range(4):
                    pa = avb[arow, pl.ds(abase + r * 16, 16)]
                    a_lo = plsc.bitcast(pa << 16, jnp.float32)
                    a_hi = plsc.bitcast(pa & jnp.int32(-65536), jnp.float32)
                    v_lo = (gsb[row, pl.ds(r * 16, 16)]
                            + gdb[row, pl.ds(r * 16, 16)] + a_lo)
                    v_hi = (gsb[row, pl.ds(64 + r * 16, 16)]
                            + gdb[row, pl.ds(64 + r * 16, 16)] + a_hi)
                    t = (jnp.maximum(v_lo, 0.0) * w2regs[r]
                         + jnp.maximum(v_hi, 0.0) * w2regs[r + 4])
                    acc = t if acc is None else acc + t
                accs.append(acc)
            cur = accs
            for k in (1, 2, 4):
                xk = lane ^ k
                pair = []
                for i2 in range(len(cur) // 2):
                    a1, b1 = cur[2 * i2], cur[2 * i2 + 1]
                    t1 = a1 + _vshuffle(a1, xk)
                    t2 = b1 + _vshuffle(b1, xk)
                    pair.append(jnp.where((lane & k) == 0, t1, t2))
                cur = pair
            h = cur[0]

            @pl.when((i & 1) == 1)
            def _():
                xk8 = lane ^ 8
                t1 = h_prev + _vshuffle(h_prev, xk8)
                t2 = h + _vshuffle(h, xk8)
                s = jnp.where((lane & 8) == 0, t1, t2) + be2
                p = 1.0 / (1.0 + jnp.exp(-s))

                gi = plsc.load_gather(si_all,
                                      [c * CHUNK + (i - 1) * 8 + lane])
                idx_s, p_s = plsc.sort_key_val(gi, p)
                for k in (1, 2, 4, 8):
                    sl = jnp.maximum(lane - k, 0)
                    pi = _vshuffle(idx_s, sl)
                    pp = _vshuffle(p_s, sl)
                    p_s = jnp.where(pi == idx_s, jnp.maximum(p_s, pp), p_s)
                nxt = _vshuffle(idx_s, jnp.minimum(lane + 1, 15))
                is_last = (idx_s != nxt) | (lane == 15)

                old = plsc.load_gather(t_v, [idx_s], mask=is_last)
                newv = jnp.maximum(old, p_s)
                plsc.store_scatter(t_v, [idx_s], newv, mask=is_last)
            return h
        lax.fori_loop(0, 2 * GROUPS, half_body, zeros16)

    def step(c, b):
        nb = 1 - b

        @pl.when(c + 1 < nchunk)
        def _():
            for cp in dat_copies(c + 1, nb):
                cp.start()
        for cp in dat_copies(c, b):
            cp.wait()
        compute(c, b)

    for cp in dat_copies(0, 0):
        cp.start()

    def pair_body(t, carry):
        step(2 * t, 0)
        step(2 * t + 1, 1)
        return carry
    lax.fori_loop(0, nchunk // 2, pair_body, 0)
    if nchunk % 2:
        step(nchunk - 1, 0)

    pltpu.sync_copy(t_v, out_hbm.at[wid])
  return _sc_edge_body


def _sc_edge(xs, xd, a, src, dst, w2, be2_pad, n_edges):
    epw = n_edges // NW
    nchunk = epw // CHUNK
    mesh = plsc.VectorSubcoreMesh(core_axis_name="c", subcore_axis_name="s")
    f = functools.partial(
        pl.kernel,
        mesh=mesh,
        compiler_params=pltpu.CompilerParams(needs_layout_passes=False),
        out_type=jax.ShapeDtypeStruct((NW, N_NODES), jnp.float32),
        scratch_types=[
            pltpu.VMEM((epw,), jnp.int32),
            pltpu.VMEM((epw,), jnp.int32),
            pltpu.VMEM((CHUNK, D), jnp.float32),
            pltpu.VMEM((CHUNK, D), jnp.float32),
            pltpu.VMEM((CHUNK, D), jnp.float32),
            pltpu.VMEM((CHUNK, D), jnp.float32),
            pltpu.VMEM((CHUNK // 2, D), jnp.int32),
            pltpu.VMEM((CHUNK // 2, D), jnp.int32),
            pltpu.VMEM((N_NODES,), jnp.float32),
            pltpu.VMEM((D,), jnp.float32),
            pltpu.VMEM((16,), jnp.float32),
            pltpu.SemaphoreType.DMA,
            pltpu.SemaphoreType.DMA,
        ],
    )(_make_sc_edge_body(epw, nchunk))
    return f(xs, xd, a, src, dst, w2, be2_pad)


def _final_body(x_ref, wn1_ref, bn1_ref, wn2_ref, bn2_ref, p1_ref, p2_ref,
                w_ref, out_ref):
    xb = x_ref[...]
    h = jnp.maximum(
        jnp.dot(xb, wn1_ref[...], preferred_element_type=jnp.float32)
        + bn1_ref[...], 0.0)
    nl = jnp.dot(h, wn2_ref[...], preferred_element_type=jnp.float32) \
        + bn2_ref[...]
    sig = jnp.maximum(jnp.max(p1_ref[...], axis=0),
                      jnp.max(p2_ref[...], axis=0))
    w = w_ref[0, 0]
    col = lax.broadcasted_iota(jnp.int32, nl.shape, 1)
    out_ref[...] = nl + jnp.where(col == 1, w * sig[:, None], 0.0)


def _final(x, wn1, bn1_row, wn2, bn2_row, part1, part2, wcomb):
    return pl.pallas_call(
        _final_body,
        out_shape=jax.ShapeDtypeStruct((N_NODES, D_OUT), jnp.float32),
    )(x, wn1, bn1_row, wn2, bn2_row, part1, part2, wcomb)


def kernel(x, edge_index, edge_attr, Wn1, bn1, Wn2, bn2, We1, be1, We2, be2,
           edge_combine_weight):
    src = edge_index[0].astype(jnp.int32)
    dst = edge_index[1].astype(jnp.int32)

    xs, xd = _node_tables(x, We1[:D], We1[D:2 * D])
    ea2 = edge_attr.reshape(N_EDGES // 2, 2 * D)
    wa = We1[2 * D:]
    b1 = be1.reshape(1, D)
    w2 = We2.reshape(D)
    be2_pad = jnp.concatenate([be2, jnp.zeros((15,), jnp.float32)])

    e_split = 192000
    a1 = _edge_hidden(ea2[:e_split // 2], wa, b1)
    part1 = _sc_edge(xs, xd, a1, src[:e_split], dst[:e_split], w2, be2_pad,
                     e_split)
    a2 = _edge_hidden(ea2[e_split // 2:], wa, b1)
    part2 = _sc_edge(xs, xd, a2, src[e_split:], dst[e_split:], w2, be2_pad,
                     N_EDGES - e_split)

    return _final(x, Wn1, bn1.reshape(1, D), Wn2, bn2.reshape(1, D_OUT),
                  part1, part2, edge_combine_weight.reshape(1, 1))


# speedup vs baseline: 1.3022x; 1.3022x over previous
import functools

import jax
import jax.numpy as jnp
from jax import lax
from jax.experimental import pallas as pl
from jax.experimental.pallas import tpu as pltpu
from jax.experimental.pallas import tpu_sc as plsc

N_NODES = 10000
N_EDGES = 320000
D = 128
D_OUT = 2

NC = 2
NS = 16
NW = NC * NS
EPW = N_EDGES // NW
CHUNK = 80
NCHUNK = EPW // CHUNK
GROUPS = CHUNK // 16

_LANE = None


def _vshuffle(v, idx):
    return lax.gather(
        v, idx[:, None],
        dimension_numbers=lax.GatherDimensionNumbers(
            offset_dims=(), collapsed_slice_dims=(0,), start_index_map=(0,)),
        slice_sizes=(1,),
        mode=lax.GatherScatterMode.PROMISE_IN_BOUNDS)


def _pack_bf16(r):
    rb = r.astype(jnp.bfloat16)
    lo = lax.bitcast_convert_type(rb[:, :D // 2], jnp.uint16).astype(jnp.uint32)
    hi = lax.bitcast_convert_type(rb[:, D // 2:], jnp.uint16).astype(jnp.uint32)
    return lax.bitcast_convert_type(lo | (hi << 16), jnp.int32)


def _node_tables_body(x_ref, ws_ref, wd_ref, xs_ref, xd_ref):
    xb = x_ref[...]
    xs_ref[...] = jnp.dot(xb, ws_ref[...], preferred_element_type=jnp.float32)
    xd_ref[...] = jnp.dot(xb, wd_ref[...], preferred_element_type=jnp.float32)


def _node_tables(x, ws, wd):
    bn = 2000
    grid = N_NODES // bn
    return pl.pallas_call(
        _node_tables_body,
        grid=(grid,),
        in_specs=[
            pl.BlockSpec((bn, D), lambda i: (i, 0)),
            pl.BlockSpec((D, D), lambda i: (0, 0)),
            pl.BlockSpec((D, D), lambda i: (0, 0)),
        ],
        out_specs=[
            pl.BlockSpec((bn, D), lambda i: (i, 0)),
            pl.BlockSpec((bn, D), lambda i: (i, 0)),
        ],
        out_shape=[
            jax.ShapeDtypeStruct((N_NODES, D), jnp.float32),
            jax.ShapeDtypeStruct((N_NODES, D), jnp.float32),
        ],
    )(x, ws, wd)


def _edge_hidden_body(ea_ref, wa_ref, b_ref, a_ref):
    ea2 = ea_ref[...]
    h0 = jnp.dot(ea2[:, :D], wa_ref[...],
                 preferred_element_type=jnp.float32) + b_ref[...]
    h1 = jnp.dot(ea2[:, D:], wa_ref[...],
                 preferred_element_type=jnp.float32) + b_ref[...]
    a_ref[...] = jnp.concatenate([_pack_bf16(h0), _pack_bf16(h1)], axis=1)


def _edge_hidden(edge_attr2, wa, be1_row):
    bm = 2000
    nrows = edge_attr2.shape[0]
    grid = nrows // bm
    return pl.pallas_call(
        _edge_hidden_body,
        grid=(grid,),
        in_specs=[
            pl.BlockSpec((bm, 2 * D), lambda i: (i, 0)),
            pl.BlockSpec((D, D), lambda i: (0, 0)),
            pl.BlockSpec((1, D), lambda i: (0, 0)),
        ],
        out_specs=pl.BlockSpec((bm, D), lambda i: (i, 0)),
        out_shape=jax.ShapeDtypeStruct((nrows, D), jnp.int32),
    )(edge_attr2, wa, be1_row)


def _make_sc_edge_body(epw, nchunk):
  def _sc_edge_body(xs_hbm, xd_hbm, a_hbm, src_hbm, dst_hbm, w2_hbm, be2_hbm,
                  out_hbm,
                  si_all, di_all, gs0, gs1, gd0, gd1,
                  av0, av1,
                  t_v, w2_v, be2_v, sem_d0, sem_d1):
    wid = lax.axis_index("s") * NC + lax.axis_index("c")
    base = wid * epw
    gs = (gs0, gs1)
    gd = (gd0, gd1)
    av = (av0, av1)
    semd = (sem_d0, sem_d1)

    pltpu.sync_copy(src_hbm.at[pl.ds(base, epw)], si_all)
    pltpu.sync_copy(dst_hbm.at[pl.ds(base, epw)], di_all)
    pltpu.sync_copy(w2_hbm, w2_v)
    pltpu.sync_copy(be2_hbm, be2_v)
    be2 = be2_v[...][0]
    w2regs = [w2_v[pl.ds(r * 16, 16)] for r in range(8)]

    lane = lax.iota(jnp.int32, 16)
    zeros16 = jnp.zeros((16,), jnp.float32)

    def zero_body(i, carry):
        t_v[pl.ds(i * 16, 16)] = zeros16
        return carry
    lax.fori_loop(0, N_NODES // 16, zero_body, 0)

    def dat_copies(c, b):
        loc = c * CHUNK
        aoff = wid * (epw // 2) + c * (CHUNK // 2)
        return (
            pltpu.make_async_copy(xs_hbm.at[si_all.at[pl.ds(loc, CHUNK)]],
                                  gs[b], semd[b]),
            pltpu.make_async_copy(xd_hbm.at[di_all.at[pl.ds(loc, CHUNK)]],
                                  gd[b], semd[b]),
            pltpu.make_async_copy(a_hbm.at[pl.ds(aoff, CHUNK // 2)],
                                  av[b], semd[b]),
        )

    def compute(c, b):
        gsb, gdb, avb = gs[b], gd[b], av[b]

        def half_body(i, h_prev):
            accs = []
            for e in range(8):
                row = i * 8 + e
                arow = i * 4 + e // 2
                abase = (e % 2) * 64
                acc = None
                for r in range(4):
                    pa = avb[arow, pl.ds(abase + r * 16, 16)]
                    a_lo = plsc.bitcast(pa << 16, jnp.float32)
                    a_hi = plsc.bitcast(pa & jnp.int32(-65536), jnp.float32)
                    v_lo = (gsb[row, pl.ds(r * 16, 16)]
                            + gdb[row, pl.ds(r * 16, 16)] + a_lo)
                    v_hi = (gsb[row, pl.ds(64 + r * 16, 16)]
                            + gdb[row, pl.ds(64 + r * 16, 16)] + a_hi)
                    t = (jnp.maximum(v_lo, 0.0) * w2regs[r]
                         + jnp.maximum(v_hi, 0.0) * w2regs[r + 4])
                    acc = t if acc is None else acc + t
                accs.append(acc)
            cur = accs
            for k in (1, 2, 4):
                xk = lane ^ k
                pair = []
                for i2 in range(len(cur) // 2):
                    a1, b1 = cur[2 * i2], cur[2 * i2 + 1]
                    t1 = a1 + _vshuffle(a1, xk)
                    t2 = b1 + _vshuffle(b1, xk)
                    pair.append(jnp.where((lane & k) == 0, t1, t2))
                cur = pair
            h = cur[0]

            @pl.when((i & 1) == 1)
            def _():
                xk8 = lane ^ 8
                t1 = h_prev + _vshuffle(h_prev, xk8)
                t2 = h + _vshuffle(h, xk8)
                s = jnp.where((lane & 8) == 0, t1, t2) + be2
                p = 1.0 / (1.0 + jnp.exp(-s))

                gi = plsc.load_gather(si_all,
                                      [c * CHUNK + (i - 1) * 8 + lane])
                idx_s, p_s = plsc.sort_key_val(gi, p)
                for k in (1, 2, 4, 8):
                    sl = jnp.maximum(lane - k, 0)
                    pi = _vshuffle(idx_s, sl)
                    pp = _vshuffle(p_s, sl)
                    p_s = jnp.where(pi == idx_s, jnp.maximum(p_s, pp), p_s)
                nxt = _vshuffle(idx_s, jnp.minimum(lane + 1, 15))
                is_last = (idx_s != nxt) | (lane == 15)

                old = plsc.load_gather(t_v, [idx_s], mask=is_last)
                newv = jnp.maximum(old, p_s)
                plsc.store_scatter(t_v, [idx_s], newv, mask=is_last)
            return h
        lax.fori_loop(0, 2 * GROUPS, half_body, zeros16)

    def step(c, b):
        nb = 1 - b

        @pl.when(c + 1 < nchunk)
        def _():
            for cp in dat_copies(c + 1, nb):
                cp.start()
        for cp in dat_copies(c, b):
            cp.wait()
        compute(c, b)

    for cp in dat_copies(0, 0):
        cp.start()

    def pair_body(t, carry):
        step(2 * t, 0)
        step(2 * t + 1, 1)
        return carry
    lax.fori_loop(0, nchunk // 2, pair_body, 0)
    if nchunk % 2:
        step(nchunk - 1, 0)

    pltpu.sync_copy(t_v, out_hbm.at[wid])
  return _sc_edge_body


def _sc_edge(xs, xd, a, src, dst, w2, be2_pad, n_edges):
    epw = n_edges // NW
    nchunk = epw // CHUNK
    mesh = plsc.VectorSubcoreMesh(core_axis_name="c", subcore_axis_name="s")
    f = functools.partial(
        pl.kernel,
        mesh=mesh,
        compiler_params=pltpu.CompilerParams(needs_layout_passes=False),
        out_type=jax.ShapeDtypeStruct((NW, N_NODES), jnp.float32),
        scratch_types=[
            pltpu.VMEM((epw,), jnp.int32),
            pltpu.VMEM((epw,), jnp.int32),
            pltpu.VMEM((CHUNK, D), jnp.float32),
            pltpu.VMEM((CHUNK, D), jnp.float32),
            pltpu.VMEM((CHUNK, D), jnp.float32),
            pltpu.VMEM((CHUNK, D), jnp.float32),
            pltpu.VMEM((CHUNK // 2, D), jnp.int32),
            pltpu.VMEM((CHUNK // 2, D), jnp.int32),
            pltpu.VMEM((N_NODES,), jnp.float32),
            pltpu.VMEM((D,), jnp.float32),
            pltpu.VMEM((16,), jnp.float32),
            pltpu.SemaphoreType.DMA,
            pltpu.SemaphoreType.DMA,
        ],
    )(_make_sc_edge_body(epw, nchunk))
    return f(xs, xd, a, src, dst, w2, be2_pad)


def _final_body(x_ref, wn1_ref, bn1_ref, wn2_ref, bn2_ref, part_ref,
                w_ref, out_ref):
    xb = x_ref[...]
    h = jnp.maximum(
        jnp.dot(xb, wn1_ref[...], preferred_element_type=jnp.float32)
        + bn1_ref[...], 0.0)
    nl = jnp.dot(h, wn2_ref[...], preferred_element_type=jnp.float32) \
        + bn2_ref[...]
    sig = jnp.max(part_ref[...], axis=0)
    w = w_ref[0, 0]
    col = lax.broadcasted_iota(jnp.int32, nl.shape, 1)
    out_ref[...] = nl + jnp.where(col == 1, w * sig[:, None], 0.0)


def _final(x, wn1, bn1_row, wn2, bn2_row, part, wcomb):
    return pl.pallas_call(
        _final_body,
        out_shape=jax.ShapeDtypeStruct((N_NODES, D_OUT), jnp.float32),
    )(x, wn1, bn1_row, wn2, bn2_row, part, wcomb)


def kernel(x, edge_index, edge_attr, Wn1, bn1, Wn2, bn2, We1, be1, We2, be2,
           edge_combine_weight):
    src = edge_index[0].astype(jnp.int32)
    dst = edge_index[1].astype(jnp.int32)

    xs, xd = _node_tables(x, We1[:D], We1[D:2 * D])
    ea2 = edge_attr.reshape(N_EDGES // 2, 2 * D)
    wa = We1[2 * D:]
    b1 = be1.reshape(1, D)
    w2 = We2.reshape(D)
    be2_pad = jnp.concatenate([be2, jnp.zeros((15,), jnp.float32)])

    a = _edge_hidden(ea2, wa, b1)
    part = _sc_edge(xs, xd, a, src, dst, w2, be2_pad, N_EDGES)

    return _final(x, Wn1, bn1.reshape(1, D), Wn2, bn2.reshape(1, D_OUT),
                  part, edge_combine_weight.reshape(1, 1))


# speedup vs baseline: 1.3392x; 1.0284x over previous
import functools

import jax
import jax.numpy as jnp
from jax import lax
from jax.experimental import pallas as pl
from jax.experimental.pallas import tpu as pltpu
from jax.experimental.pallas import tpu_sc as plsc

N_NODES = 10000
N_EDGES = 320000
D = 128
D_OUT = 2

NC = 2
NS = 16
NW = NC * NS
EPW = N_EDGES // NW
CHUNK = 80
NCHUNK = EPW // CHUNK
GROUPS = CHUNK // 16

_LANE = None


def _vshuffle(v, idx):
    return lax.gather(
        v, idx[:, None],
        dimension_numbers=lax.GatherDimensionNumbers(
            offset_dims=(), collapsed_slice_dims=(0,), start_index_map=(0,)),
        slice_sizes=(1,),
        mode=lax.GatherScatterMode.PROMISE_IN_BOUNDS)


def _pack_bf16(r):
    rb = r.astype(jnp.bfloat16)
    lo = lax.bitcast_convert_type(rb[:, :D // 2], jnp.uint16).astype(jnp.uint32)
    hi = lax.bitcast_convert_type(rb[:, D // 2:], jnp.uint16).astype(jnp.uint32)
    return lax.bitcast_convert_type(lo | (hi << 16), jnp.int32)


def _node_tables_body(x_ref, ws_ref, wd_ref, xs_ref, xd_ref):
    xb = x_ref[...]
    xs_ref[...] = jnp.dot(xb, ws_ref[...], preferred_element_type=jnp.float32)
    xd_ref[...] = jnp.dot(xb, wd_ref[...], preferred_element_type=jnp.float32)


def _node_tables(x, ws, wd):
    bn = 2000
    grid = N_NODES // bn
    return pl.pallas_call(
        _node_tables_body,
        grid=(grid,),
        in_specs=[
            pl.BlockSpec((bn, D), lambda i: (i, 0)),
            pl.BlockSpec((D, D), lambda i: (0, 0)),
            pl.BlockSpec((D, D), lambda i: (0, 0)),
        ],
        out_specs=[
            pl.BlockSpec((bn, D), lambda i: (i, 0)),
            pl.BlockSpec((bn, D), lambda i: (i, 0)),
        ],
        out_shape=[
            jax.ShapeDtypeStruct((N_NODES, D), jnp.float32),
            jax.ShapeDtypeStruct((N_NODES, D), jnp.float32),
        ],
    )(x, ws, wd)


def _edge_hidden_body(ea_ref, wa_ref, b_ref, a_ref):
    ea2 = ea_ref[...]
    h0 = jnp.dot(ea2[:, :D], wa_ref[...],
                 preferred_element_type=jnp.float32) + b_ref[...]
    h1 = jnp.dot(ea2[:, D:], wa_ref[...],
                 preferred_element_type=jnp.float32) + b_ref[...]
    a_ref[...] = jnp.concatenate([_pack_bf16(h0), _pack_bf16(h1)], axis=1)


def _edge_hidden(edge_attr2, wa, be1_row):
    bm = 2000
    nrows = edge_attr2.shape[0]
    grid = nrows // bm
    return pl.pallas_call(
        _edge_hidden_body,
        grid=(grid,),
        in_specs=[
            pl.BlockSpec((bm, 2 * D), lambda i: (i, 0)),
            pl.BlockSpec((D, D), lambda i: (0, 0)),
            pl.BlockSpec((1, D), lambda i: (0, 0)),
        ],
        out_specs=pl.BlockSpec((bm, D), lambda i: (i, 0)),
        out_shape=jax.ShapeDtypeStruct((nrows, D), jnp.int32),
    )(edge_attr2, wa, be1_row)


def _make_sc_edge_body(epw, nchunk):
  def _sc_edge_body(xs_hbm, xd_hbm, a_hbm, src_hbm, dst_hbm, w2_hbm, be2_hbm,
                  out_hbm,
                  si_all, di_all, gs0, gs1, gs2, gd0, gd1, gd2,
                  av0, av1, av2,
                  t_v, w2_v, be2_v, sem_d0, sem_d1, sem_d2):
    wid = lax.axis_index("s") * NC + lax.axis_index("c")
    base = wid * epw
    gs = (gs0, gs1, gs2)
    gd = (gd0, gd1, gd2)
    av = (av0, av1, av2)
    semd = (sem_d0, sem_d1, sem_d2)

    pltpu.sync_copy(src_hbm.at[pl.ds(base, epw)], si_all)
    pltpu.sync_copy(dst_hbm.at[pl.ds(base, epw)], di_all)
    pltpu.sync_copy(w2_hbm, w2_v)
    pltpu.sync_copy(be2_hbm, be2_v)
    be2 = be2_v[...][0]
    w2regs = [w2_v[pl.ds(r * 16, 16)] for r in range(8)]

    lane = lax.iota(jnp.int32, 16)
    zeros16 = jnp.zeros((16,), jnp.float32)

    def zero_body(i, carry):
        t_v[pl.ds(i * 16, 16)] = zeros16
        return carry
    lax.fori_loop(0, N_NODES // 16, zero_body, 0)

    def dat_copies(c, b):
        loc = c * CHUNK
        aoff = wid * (epw // 2) + c * (CHUNK // 2)
        return (
            pltpu.make_async_copy(xs_hbm.at[si_all.at[pl.ds(loc, CHUNK)]],
                                  gs[b], semd[b]),
            pltpu.make_async_copy(xd_hbm.at[di_all.at[pl.ds(loc, CHUNK)]],
                                  gd[b], semd[b]),
            pltpu.make_async_copy(a_hbm.at[pl.ds(aoff, CHUNK // 2)],
                                  av[b], semd[b]),
        )

    def compute(c, b):
        gsb, gdb, avb = gs[b], gd[b], av[b]

        def half_body(i, h_prev):
            accs = []
            for e in range(8):
                row = i * 8 + e
                arow = i * 4 + e // 2
                abase = (e % 2) * 64
                acc = None
                for r in range(4):
                    pa = avb[arow, pl.ds(abase + r * 16, 16)]
                    a_lo = plsc.bitcast(pa << 16, jnp.float32)
                    a_hi = plsc.bitcast(pa & jnp.int32(-65536), jnp.float32)
                    v_lo = (gsb[row, pl.ds(r * 16, 16)]
                            + gdb[row, pl.ds(r * 16, 16)] + a_lo)
                    v_hi = (gsb[row, pl.ds(64 + r * 16, 16)]
                            + gdb[row, pl.ds(64 + r * 16, 16)] + a_hi)
                    t = (jnp.maximum(v_lo, 0.0) * w2regs[r]
                         + jnp.maximum(v_hi, 0.0) * w2regs[r + 4])
                    acc = t if acc is None else acc + t
                accs.append(acc)
            cur = accs
            for k in (1, 2, 4):
                xk = lane ^ k
                pair = []
                for i2 in range(len(cur) // 2):
                    a1, b1 = cur[2 * i2], cur[2 * i2 + 1]
                    t1 = a1 + _vshuffle(a1, xk)
                    t2 = b1 + _vshuffle(b1, xk)
                    pair.append(jnp.where((lane & k) == 0, t1, t2))
                cur = pair
            h = cur[0]

            @pl.when((i & 1) == 1)
            def _():
                xk8 = lane ^ 8
                t1 = h_prev + _vshuffle(h_prev, xk8)
                t2 = h + _vshuffle(h, xk8)
                s = jnp.where((lane & 8) == 0, t1, t2) + be2
                p = 1.0 / (1.0 + jnp.exp(-s))

                gi = plsc.load_gather(si_all,
                                      [c * CHUNK + (i - 1) * 8 + lane])
                idx_s, p_s = plsc.sort_key_val(gi, p)
                for k in (1, 2, 4, 8):
                    sl = jnp.maximum(lane - k, 0)
                    pi = _vshuffle(idx_s, sl)
                    pp = _vshuffle(p_s, sl)
                    p_s = jnp.where(pi == idx_s, jnp.maximum(p_s, pp), p_s)
                nxt = _vshuffle(idx_s, jnp.minimum(lane + 1, 15))
                is_last = (idx_s != nxt) | (lane == 15)

                old = plsc.load_gather(t_v, [idx_s], mask=is_last)
                newv = jnp.maximum(old, p_s)
                plsc.store_scatter(t_v, [idx_s], newv, mask=is_last)
            return h
        lax.fori_loop(0, 2 * GROUPS, half_body, zeros16)

    def step(c, b):
        nb = (b + 2) % 3

        @pl.when(c + 2 < nchunk)
        def _():
            for cp in dat_copies(c + 2, nb):
                cp.start()
        for cp in dat_copies(c, b):
            cp.wait()
        compute(c, b)

    for cp in dat_copies(0, 0):
        cp.start()
    for cp in dat_copies(1, 1):
        cp.start()

    ntriple = (nchunk - 2) // 3

    def triple_body(t, carry):
        step(3 * t, 0)
        step(3 * t + 1, 1)
        step(3 * t + 2, 2)
        return carry
    lax.fori_loop(0, ntriple, triple_body, 0)
    for c in range(3 * ntriple, nchunk):
        step(c, c % 3)

    pltpu.sync_copy(t_v, out_hbm.at[wid])
  return _sc_edge_body


def _sc_edge(xs, xd, a, src, dst, w2, be2_pad, n_edges):
    epw = n_edges // NW
    nchunk = epw // CHUNK
    mesh = plsc.VectorSubcoreMesh(core_axis_name="c", subcore_axis_name="s")
    f = functools.partial(
        pl.kernel,
        mesh=mesh,
        compiler_params=pltpu.CompilerParams(needs_layout_passes=False),
        out_type=jax.ShapeDtypeStruct((NW, N_NODES), jnp.float32),
        scratch_types=[
            pltpu.VMEM((epw,), jnp.int32),
            pltpu.VMEM((epw,), jnp.int32),
            pltpu.VMEM((CHUNK, D), jnp.float32),
            pltpu.VMEM((CHUNK, D), jnp.float32),
            pltpu.VMEM((CHUNK, D), jnp.float32),
            pltpu.VMEM((CHUNK, D), jnp.float32),
            pltpu.VMEM((CHUNK, D), jnp.float32),
            pltpu.VMEM((CHUNK, D), jnp.float32),
            pltpu.VMEM((CHUNK // 2, D), jnp.int32),
            pltpu.VMEM((CHUNK // 2, D), jnp.int32),
            pltpu.VMEM((CHUNK // 2, D), jnp.int32),
            pltpu.VMEM((N_NODES,), jnp.float32),
            pltpu.VMEM((D,), jnp.float32),
            pltpu.VMEM((16,), jnp.float32),
            pltpu.SemaphoreType.DMA,
            pltpu.SemaphoreType.DMA,
            pltpu.SemaphoreType.DMA,
        ],
    )(_make_sc_edge_body(epw, nchunk))
    return f(xs, xd, a, src, dst, w2, be2_pad)


def _final_body(x_ref, wn1_ref, bn1_ref, wn2_ref, bn2_ref, part_ref,
                w_ref, out_ref):
    xb = x_ref[...]
    h = jnp.maximum(
        jnp.dot(xb, wn1_ref[...], preferred_element_type=jnp.float32)
        + bn1_ref[...], 0.0)
    nl = jnp.dot(h, wn2_ref[...], preferred_element_type=jnp.float32) \
        + bn2_ref[...]
    sig = jnp.max(part_ref[...], axis=0)
    w = w_ref[0, 0]
    col = lax.broadcasted_iota(jnp.int32, nl.shape, 1)
    out_ref[...] = nl + jnp.where(col == 1, w * sig[:, None], 0.0)


def _final(x, wn1, bn1_row, wn2, bn2_row, part, wcomb):
    return pl.pallas_call(
        _final_body,
        out_shape=jax.ShapeDtypeStruct((N_NODES, D_OUT), jnp.float32),
    )(x, wn1, bn1_row, wn2, bn2_row, part, wcomb)


def kernel(x, edge_index, edge_attr, Wn1, bn1, Wn2, bn2, We1, be1, We2, be2,
           edge_combine_weight):
    src = edge_index[0].astype(jnp.int32)
    dst = edge_index[1].astype(jnp.int32)

    xs, xd = _node_tables(x, We1[:D], We1[D:2 * D])
    ea2 = edge_attr.reshape(N_EDGES // 2, 2 * D)
    wa = We1[2 * D:]
    b1 = be1.reshape(1, D)
    w2 = We2.reshape(D)
    be2_pad = jnp.concatenate([be2, jnp.zeros((15,), jnp.float32)])

    a = _edge_hidden(ea2, wa, b1)
    part = _sc_edge(xs, xd, a, src, dst, w2, be2_pad, N_EDGES)

    return _final(x, Wn1, bn1.reshape(1, D), Wn2, bn2.reshape(1, D_OUT),
                  part, edge_combine_weight.reshape(1, 1))


# speedup vs baseline: 1.3406x; 1.0011x over previous
import functools

import jax
import jax.numpy as jnp
from jax import lax
from jax.experimental import pallas as pl
from jax.experimental.pallas import tpu as pltpu
from jax.experimental.pallas import tpu_sc as plsc

N_NODES = 10000
N_EDGES = 320000
D = 128
D_OUT = 2

NC = 2
NS = 16
NW = NC * NS
CHUNK = 80
GROUPS = CHUNK // 16


def _vshuffle(v, idx):
    return lax.gather(
        v, idx[:, None],
        dimension_numbers=lax.GatherDimensionNumbers(
            offset_dims=(), collapsed_slice_dims=(0,), start_index_map=(0,)),
        slice_sizes=(1,),
        mode=lax.GatherScatterMode.PROMISE_IN_BOUNDS)


def _pack_bf16(r):
    rb = r.astype(jnp.bfloat16)
    lo = lax.bitcast_convert_type(rb[:, :D // 2], jnp.uint16).astype(jnp.uint32)
    hi = lax.bitcast_convert_type(rb[:, D // 2:], jnp.uint16).astype(jnp.uint32)
    return lax.bitcast_convert_type(lo | (hi << 16), jnp.int32)


def _node_tables_body(x_ref, ws_ref, wd_ref, xs_ref, xd_ref):
    xb = x_ref[...]
    xs_ref[...] = jnp.dot(xb, ws_ref[...], preferred_element_type=jnp.float32)
    xd_ref[...] = jnp.dot(xb, wd_ref[...], preferred_element_type=jnp.float32)


def _node_tables(x, ws, wd):
    bn = 2000
    grid = N_NODES // bn
    return pl.pallas_call(
        _node_tables_body,
        grid=(grid,),
        in_specs=[
            pl.BlockSpec((bn, D), lambda i: (i, 0)),
            pl.BlockSpec((D, D), lambda i: (0, 0)),
            pl.BlockSpec((D, D), lambda i: (0, 0)),
        ],
        out_specs=[
            pl.BlockSpec((bn, D), lambda i: (i, 0)),
            pl.BlockSpec((bn, D), lambda i: (i, 0)),
        ],
        out_shape=[
            jax.ShapeDtypeStruct((N_NODES, D), jnp.float32),
            jax.ShapeDtypeStruct((N_NODES, D), jnp.float32),
        ],
    )(x, ws, wd)


def _edge_hidden_body(ea_ref, wa_ref, b_ref, a_ref):
    ea2 = ea_ref[...]
    h0 = jnp.dot(ea2[:, :D], wa_ref[...],
                 preferred_element_type=jnp.float32) + b_ref[...]
    h1 = jnp.dot(ea2[:, D:], wa_ref[...],
                 preferred_element_type=jnp.float32) + b_ref[...]
    a_ref[...] = jnp.concatenate([_pack_bf16(h0), _pack_bf16(h1)], axis=1)


def _edge_hidden(edge_attr2, wa, be1_row):
    bm = 2000
    nrows = edge_attr2.shape[0]
    grid = nrows // bm
    return pl.pallas_call(
        _edge_hidden_body,
        grid=(grid,),
        in_specs=[
            pl.BlockSpec((bm, 2 * D), lambda i: (i, 0)),
            pl.BlockSpec((D, D), lambda i: (0, 0)),
            pl.BlockSpec((1, D), lambda i: (0, 0)),
        ],
        out_specs=pl.BlockSpec((bm, D), lambda i: (i, 0)),
        out_shape=jax.ShapeDtypeStruct((nrows, D), jnp.int32),
    )(edge_attr2, wa, be1_row)


def _make_sc_edge_body(epw, nchunk):
  def _sc_edge_body(xs_hbm, xd_hbm, a_hbm, src_hbm, dst_hbm, w2_hbm, be2_hbm,
                  out_hbm,
                  si_all, di_all, gs0, gs1, gs2, gd0, gd1, gd2,
                  av0, av1, av2,
                  t_v, w2_v, be2_v, sem_d0, sem_d1, sem_d2):
    wid = lax.axis_index("s") * NC + lax.axis_index("c")
    base = wid * epw
    gs = (gs0, gs1, gs2)
    gd = (gd0, gd1, gd2)
    av = (av0, av1, av2)
    semd = (sem_d0, sem_d1, sem_d2)

    pltpu.sync_copy(src_hbm.at[pl.ds(base, epw)], si_all)
    pltpu.sync_copy(dst_hbm.at[pl.ds(base, epw)], di_all)
    pltpu.sync_copy(w2_hbm, w2_v)
    pltpu.sync_copy(be2_hbm, be2_v)
    be2 = be2_v[...][0]
    w2regs = [w2_v[pl.ds(r * 16, 16)] for r in range(8)]

    lane = lax.iota(jnp.int32, 16)
    zeros16 = jnp.zeros((16,), jnp.float32)

    def zero_body(i, carry):
        t_v[pl.ds(i * 16, 16)] = zeros16
        return carry
    lax.fori_loop(0, N_NODES // 16, zero_body, 0)

    def dat_copies(c, b):
        loc = c * CHUNK
        aoff = wid * (epw // 2) + c * (CHUNK // 2)
        return (
            pltpu.make_async_copy(xs_hbm.at[si_all.at[pl.ds(loc, CHUNK)]],
                                  gs[b], semd[b]),
            pltpu.make_async_copy(xd_hbm.at[di_all.at[pl.ds(loc, CHUNK)]],
                                  gd[b], semd[b]),
            pltpu.make_async_copy(a_hbm.at[pl.ds(aoff, CHUNK // 2)],
                                  av[b], semd[b]),
        )

    def compute(c, b):
        gsb, gdb, avb = gs[b], gd[b], av[b]

        def half_body(i, h_prev):
            accs = []
            for e in range(8):
                row = i * 8 + e
                arow = i * 4 + e // 2
                abase = (e % 2) * 64
                acc = None
                for r in range(4):
                    pa = avb[arow, pl.ds(abase + r * 16, 16)]
                    a_lo = plsc.bitcast(pa << 16, jnp.float32)
                    a_hi = plsc.bitcast(pa & jnp.int32(-65536), jnp.float32)
                    v_lo = (gsb[row, pl.ds(r * 16, 16)]
                            + gdb[row, pl.ds(r * 16, 16)] + a_lo)
                    v_hi = (gsb[row, pl.ds(64 + r * 16, 16)]
                            + gdb[row, pl.ds(64 + r * 16, 16)] + a_hi)
                    t = (jnp.maximum(v_lo, 0.0) * w2regs[r]
                         + jnp.maximum(v_hi, 0.0) * w2regs[r + 4])
                    acc = t if acc is None else acc + t
                accs.append(acc)
            cur = accs
            for k in (1, 2, 4):
                xk = lane ^ k
                pair = []
                for i2 in range(len(cur) // 2):
                    a1, b1 = cur[2 * i2], cur[2 * i2 + 1]
                    t1 = a1 + _vshuffle(a1, xk)
                    t2 = b1 + _vshuffle(b1, xk)
                    pair.append(jnp.where((lane & k) == 0, t1, t2))
                cur = pair
            h = cur[0]

            @pl.when((i & 1) == 1)
            def _():
                xk8 = lane ^ 8
                t1 = h_prev + _vshuffle(h_prev, xk8)
                t2 = h + _vshuffle(h, xk8)
                s = jnp.where((lane & 8) == 0, t1, t2) + be2
                p = 1.0 / (1.0 + jnp.exp(-s))

                gi = plsc.load_gather(si_all,
                                      [c * CHUNK + (i - 1) * 8 + lane])
                idx_s, p_s = plsc.sort_key_val(gi, p)
                for k in (1, 2, 4, 8):
                    sl = jnp.maximum(lane - k, 0)
                    pi = _vshuffle(idx_s, sl)
                    pp = _vshuffle(p_s, sl)
                    p_s = jnp.where(pi == idx_s, jnp.maximum(p_s, pp), p_s)
                nxt = _vshuffle(idx_s, jnp.minimum(lane + 1, 15))
                is_last = (idx_s != nxt) | (lane == 15)

                old = plsc.load_gather(t_v, [idx_s], mask=is_last)
                newv = jnp.maximum(old, p_s)
                plsc.store_scatter(t_v, [idx_s], newv, mask=is_last)
            return h
        lax.fori_loop(0, 2 * GROUPS, half_body, zeros16)

    def step(c, b):
        nb = (b + 2) % 3

        @pl.when(c + 2 < nchunk)
        def _():
            for cp in dat_copies(c + 2, nb):
                cp.start()
        for cp in dat_copies(c, b):
            cp.wait()
        compute(c, b)

    for cp in dat_copies(0, 0):
        cp.start()
    for cp in dat_copies(1, 1):
        cp.start()

    ntriple = (nchunk - 2) // 3

    def triple_body(t, carry):
        step(3 * t, 0)
        step(3 * t + 1, 1)
        step(3 * t + 2, 2)
        return carry
    lax.fori_loop(0, ntriple, triple_body, 0)
    for c in range(3 * ntriple, nchunk):
        step(c, c % 3)

    pltpu.sync_copy(t_v, out_hbm.at[wid])
  return _sc_edge_body


def _sc_edge(xs, xd, a, src, dst, w2, be2_pad, n_edges):
    epw = n_edges // NW
    nchunk = epw // CHUNK
    mesh = plsc.VectorSubcoreMesh(core_axis_name="c", subcore_axis_name="s")
    f = functools.partial(
        pl.kernel,
        mesh=mesh,
        compiler_params=pltpu.CompilerParams(needs_layout_passes=False),
        out_type=jax.ShapeDtypeStruct((NW, N_NODES), jnp.float32),
        scratch_types=[
            pltpu.VMEM((epw,), jnp.int32),
            pltpu.VMEM((epw,), jnp.int32),
            pltpu.VMEM((CHUNK, D), jnp.float32),
            pltpu.VMEM((CHUNK, D), jnp.float32),
            pltpu.VMEM((CHUNK, D), jnp.float32),
            pltpu.VMEM((CHUNK, D), jnp.float32),
            pltpu.VMEM((CHUNK, D), jnp.float32),
            pltpu.VMEM((CHUNK, D), jnp.float32),
            pltpu.VMEM((CHUNK // 2, D), jnp.int32),
            pltpu.VMEM((CHUNK // 2, D), jnp.int32),
            pltpu.VMEM((CHUNK // 2, D), jnp.int32),
            pltpu.VMEM((N_NODES,), jnp.float32),
            pltpu.VMEM((D,), jnp.float32),
            pltpu.VMEM((16,), jnp.float32),
            pltpu.SemaphoreType.DMA,
            pltpu.SemaphoreType.DMA,
            pltpu.SemaphoreType.DMA,
        ],
    )(_make_sc_edge_body(epw, nchunk))
    return f(xs, xd, a, src, dst, w2, be2_pad)


def _final_body(x_ref, wn1_ref, bn1_ref, wn2_ref, bn2_ref, part_ref,
                w_ref, out_ref):
    xb = x_ref[...]
    h = jnp.maximum(
        jnp.dot(xb, wn1_ref[...], preferred_element_type=jnp.float32)
        + bn1_ref[...], 0.0)
    nl = jnp.dot(h, wn2_ref[...], preferred_element_type=jnp.float32) \
        + bn2_ref[...]
    sig = jnp.max(part_ref[...], axis=0)
    w = w_ref[0, 0]
    col = lax.broadcasted_iota(jnp.int32, nl.shape, 1)
    out_ref[...] = nl + jnp.where(col == 1, w * sig[:, None], 0.0)


def _final(x, wn1, bn1_row, wn2, bn2_row, part, wcomb):
    return pl.pallas_call(
        _final_body,
        out_shape=jax.ShapeDtypeStruct((N_NODES, D_OUT), jnp.float32),
    )(x, wn1, bn1_row, wn2, bn2_row, part, wcomb)


def kernel(x, edge_index, edge_attr, Wn1, bn1, Wn2, bn2, We1, be1, We2, be2,
           edge_combine_weight):
    src = edge_index[0].astype(jnp.int32)
    dst = edge_index[1].astype(jnp.int32)

    xs, xd = _node_tables(x, We1[:D], We1[D:2 * D])
    ea2 = edge_attr.reshape(N_EDGES // 2, 2 * D)
    wa = We1[2 * D:]
    b1 = be1.reshape(1, D)
    w2 = We2.reshape(D)
    be2_pad = jnp.concatenate([be2, jnp.zeros((15,), jnp.float32)])

    a = _edge_hidden(ea2, wa, b1)
    part = _sc_edge(xs, xd, a, src, dst, w2, be2_pad, N_EDGES)

    return _final(x, Wn1, bn1.reshape(1, D), Wn2, bn2.reshape(1, D_OUT),
                  part, edge_combine_weight.reshape(1, 1))
